# baseline (speedup 1.0000x reference)
import dataclasses
import functools

import jax
import jax.numpy as jnp
from jax import lax
from jax.experimental import pallas as pl
from jax.experimental.pallas import tpu as pltpu
from jax.experimental.pallas import tpu_sc as plsc

N = 16384
C = 200
NC = 2
NS = 16
NW = NC * NS
L = 16
COLS_PER_W = N // NW
CHUNK = 128
NCHUNKS = COLS_PER_W // CHUNK
NXBUF = 2
NDBUF = 3

_mesh = plsc.VectorSubcoreMesh(core_axis_name="c", subcore_axis_name="s")

_cp = pltpu.CompilerParams()
if "needs_layout_passes" in pltpu.CompilerParams.__dataclass_fields__:
  _cp = dataclasses.replace(_cp, needs_layout_passes=False)

_scratch = (
    [pltpu.VMEM((C, CHUNK), jnp.float32) for _ in range(NXBUF)]
    + [pltpu.VMEM((C, CHUNK), jnp.int32) for _ in range(NDBUF)]
    + [pltpu.SemaphoreType.DMA] * (NXBUF + 2 * NDBUF)
)


@jax.jit
def _gather_sc(xt, dt):
  @functools.partial(
      pl.kernel,
      out_type=jax.ShapeDtypeStruct((C, N), jnp.float32),
      mesh=_mesh,
      scratch_types=_scratch,
      compiler_params=_cp,
  )
  def k(x_hbm, d_hbm, o_hbm, *scr):
    xvs = scr[:NXBUF]
    dvs = scr[NXBUF:NXBUF + NDBUF]
    sxs = scr[NXBUF + NDBUF:2 * NXBUF + NDBUF]
    sds = scr[2 * NXBUF + NDBUF:2 * NXBUF + 2 * NDBUF]
    sos = scr[2 * NXBUF + 2 * NDBUF:]
    wid = lax.axis_index("s") * NC + lax.axis_index("c")
    base = wid * COLS_PER_W
    lane = lax.iota(jnp.int32, L)
    cols = [lane + o for o in range(0, CHUNK, L)]

    pend_in = {}
    pend_out = {}

    def issue_in(cc):
      c0 = base + cc * CHUNK
      pend_in[cc] = (
          pltpu.async_copy(
              x_hbm.at[:, pl.ds(c0, CHUNK)], xvs[cc % NXBUF], sxs[cc % NXBUF]),
          pltpu.async_copy(
              d_hbm.at[:, pl.ds(c0, CHUNK)], dvs[cc % NDBUF], sds[cc % NDBUF]),
      )

    for cc in range(min(NXBUF, NCHUNKS)):
      issue_in(cc)

    for cc in range(NCHUNKS):
      xv = xvs[cc % NXBUF]
      dv = dvs[cc % NDBUF]
      cpx, cpd = pend_in.pop(cc)
      cpx.wait()
      cpd.wait()

      @plsc.parallel_loop(0, C, unroll=4)
      def _(j):
        for o, col in zip(range(0, CHUNK, L), cols):
          idx = dv[j, pl.ds(o, L)]
          vals = plsc.load_gather(xv, [idx, col])
          dv[j, pl.ds(o, L)] = plsc.bitcast(vals, jnp.int32)

      c0 = base + cc * CHUNK
      pend_out[cc] = pltpu.async_copy(
          dv.bitcast(jnp.float32), o_hbm.at[:, pl.ds(c0, CHUNK)],
          sos[cc % NDBUF])
      if cc + NXBUF < NCHUNKS:
        prev = cc + NXBUF - NDBUF
        if prev >= 0:
          pend_out.pop(prev).wait()
        issue_in(cc + NXBUF)

    for cc in sorted(pend_out):
      pend_out.pop(cc).wait()

  return k(xt, dt)


def kernel(x, d):
  out_t = _gather_sc(x.T, d.astype(jnp.int32).T)
  return out_t.T


# speedup vs baseline: 1.0130x; 1.0130x over previous
import dataclasses
import functools

import jax
import jax.numpy as jnp
from jax import lax
from jax.experimental import pallas as pl
from jax.experimental.pallas import tpu as pltpu
from jax.experimental.pallas import tpu_sc as plsc

N = 16384
C = 200
NC = 2
NS = 16
NW = NC * NS
L = 16
COLS_PER_W = N // NW
CHUNK = 128
NCHUNKS = COLS_PER_W // CHUNK
NXBUF = 2
NDBUF = 3

_mesh = plsc.VectorSubcoreMesh(core_axis_name="c", subcore_axis_name="s")

_cp = pltpu.CompilerParams()
if "needs_layout_passes" in pltpu.CompilerParams.__dataclass_fields__:
  _cp = dataclasses.replace(_cp, needs_layout_passes=False)

_scratch = (
    [pltpu.VMEM((C, CHUNK), jnp.float32) for _ in range(NXBUF)]
    + [pltpu.VMEM((C, CHUNK), jnp.int32) for _ in range(NDBUF)]
    + [pltpu.SemaphoreType.DMA] * (NXBUF + 2 * NDBUF)
)


@jax.jit
def _gather_sc(xt, dt):
  @functools.partial(
      pl.kernel,
      out_type=jax.ShapeDtypeStruct((C, N), jnp.float32),
      mesh=_mesh,
      scratch_types=_scratch,
      compiler_params=_cp,
  )
  def k(x_hbm, d_hbm, o_hbm, *scr):
    xvs = scr[:NXBUF]
    dvs = scr[NXBUF:NXBUF + NDBUF]
    sxs = scr[NXBUF + NDBUF:2 * NXBUF + NDBUF]
    sds = scr[2 * NXBUF + NDBUF:2 * NXBUF + 2 * NDBUF]
    sos = scr[2 * NXBUF + 2 * NDBUF:]
    wid = lax.axis_index("s") * NC + lax.axis_index("c")
    base = wid * COLS_PER_W
    lane = lax.iota(jnp.int32, L)
    cols = [lane + o for o in range(0, CHUNK, L)]

    pend_in = {}
    pend_out = {}

    def issue_in(cc):
      c0 = base + cc * CHUNK
      pend_in[cc] = (
          pltpu.async_copy(
              x_hbm.at[:, pl.ds(c0, CHUNK)], xvs[cc % NXBUF], sxs[cc % NXBUF]),
          pltpu.async_copy(
              d_hbm.at[:, pl.ds(c0, CHUNK)], dvs[cc % NDBUF], sds[cc % NDBUF]),
      )

    for cc in range(min(NXBUF, NCHUNKS)):
      issue_in(cc)

    for cc in range(NCHUNKS):
      xv = xvs[cc % NXBUF]
      dv = dvs[cc % NDBUF]
      cpx, cpd = pend_in.pop(cc)
      cpx.wait()
      cpd.wait()

      @plsc.parallel_loop(0, C, unroll=2)
      def _(j):
        for o, col in zip(range(0, CHUNK, L), cols):
          idx = dv[j, pl.ds(o, L)]
          vals = plsc.load_gather(xv, [idx, col])
          dv[j, pl.ds(o, L)] = plsc.bitcast(vals, jnp.int32)

      c0 = base + cc * CHUNK
      pend_out[cc] = pltpu.async_copy(
          dv.bitcast(jnp.float32), o_hbm.at[:, pl.ds(c0, CHUNK)],
          sos[cc % NDBUF])
      if cc + NXBUF < NCHUNKS:
        prev = cc + NXBUF - NDBUF
        if prev >= 0:
          pend_out.pop(prev).wait()
        issue_in(cc + NXBUF)

    for cc in sorted(pend_out):
      pend_out.pop(cc).wait()

  return k(xt, dt)


def kernel(x, d):
  out_t = _gather_sc(x.T, d.astype(jnp.int32).T)
  return out_t.T
